# baseline (speedup 1.0000x reference)
import functools

import jax
import jax.numpy as jnp
from jax import lax
from jax.experimental import pallas as pl
from jax.experimental.pallas import tpu as pltpu
from jax.experimental.pallas import tpu_sc as plsc

N = 10000
NP = 10240
D = 128
DH = D // 2
E = 320000
NUM_MP = 10

NC = 2
NS = 16
B = 256
NB = 80
P = 20
NPH = NB // P
EPS = B * NB
EPAD = NS * EPS
ZR = NP // NS

BM = 2048
GRID = NP // BM

_MESH = plsc.VectorSubcoreMesh(
    core_axis_name="c", subcore_axis_name="s", num_cores=NC, num_subcores=NS)


def _sc_deg_body(send_hbm, recv_hbm, ones_hbm, zeros_hbm, out_s_hbm, out_r_hbm,
                 idx_v, ones_v, acc):
    c = lax.axis_index("c")
    s = lax.axis_index("s")

    @pl.when(c == 0)
    def _():
        pltpu.sync_copy(send_hbm.at[s], idx_v)

    @pl.when(c == 1)
    def _():
        pltpu.sync_copy(recv_hbm.at[s], idx_v)

    pltpu.sync_copy(ones_hbm, ones_v)
    pltpu.sync_copy(zeros_hbm, acc.at[pl.ds(s * ZR, ZR)])
    plsc.subcore_barrier()

    @pl.loop(0, NB)
    def _(j):
        pltpu.sync_copy(ones_v, acc.at[idx_v.at[j]], add=True)

    plsc.subcore_barrier()

    @pl.when(c == 0)
    def _():
        pltpu.sync_copy(acc.at[pl.ds(s * ZR, ZR)], out_s_hbm.at[pl.ds(s * ZR, ZR)])

    @pl.when(c == 1)
    def _():
        pltpu.sync_copy(acc.at[pl.ds(s * ZR, ZR)], out_r_hbm.at[pl.ds(s * ZR, ZR)])


_sc_deg = pl.kernel(
    _sc_deg_body,
    out_type=(jax.ShapeDtypeStruct((NP, 8), jnp.float32),
              jax.ShapeDtypeStruct((NP, 8), jnp.float32)),
    mesh=_MESH,
    scratch_types=[
        pltpu.VMEM((NB, B), jnp.int32),
        pltpu.VMEM((B, 8), jnp.float32),
        pltpu.VMEM_SHARED((NP, 8), jnp.float32),
    ],
    compiler_params=pltpu.CompilerParams(use_tc_tiling_on_sc=False),
)


def _sc_agg_body(h_hbm, send_hbm, recv_hbm, zeros_hbm, out_hbm,
                 sidx, ridx, rows_v, acc, sg0, sg1):
    gsems = (sg0, sg1)
    c = lax.axis_index("c")
    s = lax.axis_index("s")
    h_c = h_hbm.at[c]
    pltpu.sync_copy(zeros_hbm, acc.at[pl.ds(s * ZR, ZR)])
    plsc.subcore_barrier()

    for ph in range(NPH):
        pltpu.sync_copy(send_hbm.at[s, pl.ds(ph * P, P)], sidx)
        pltpu.sync_copy(recv_hbm.at[s, pl.ds(ph * P, P)], ridx)
        pltpu.async_copy(h_c.at[sidx.at[0]], rows_v.at[0], gsems[0])
        pltpu.async_copy(h_c.at[sidx.at[1]], rows_v.at[1], gsems[1])

        @pl.loop(0, P, step=2)
        def _(j0):
            for b in range(2):
                j = j0 + b
                pltpu.make_async_copy(h_c.at[sidx.at[j]], rows_v.at[b],
                                      gsems[b]).wait()
                pltpu.sync_copy(rows_v.at[b], acc.at[ridx.at[j]], add=True)

                @pl.when(j + 2 < P)
                def _():
                    pltpu.async_copy(h_c.at[sidx.at[j + 2]], rows_v.at[b],
                                     gsems[b])

    plsc.subcore_barrier()
    pltpu.sync_copy(acc.at[pl.ds(s * ZR, ZR)],
                    out_hbm.at[c, pl.ds(s * ZR, ZR)])


_sc_agg = pl.kernel(
    _sc_agg_body,
    out_type=jax.ShapeDtypeStruct((NC, NP, DH), jnp.float32),
    mesh=_MESH,
    scratch_types=[
        pltpu.VMEM((P, B), jnp.int32),
        pltpu.VMEM((P, B), jnp.int32),
        pltpu.VMEM((2, B, DH), jnp.float32),
        pltpu.VMEM_SHARED((NP, DH), jnp.float32),
        pltpu.SemaphoreType.DMA,
        pltpu.SemaphoreType.DMA,
    ],
    compiler_params=pltpu.CompilerParams(use_tc_tiling_on_sc=False),
)


_DOT = functools.partial(jnp.dot, preferred_element_type=jnp.float32,
                         precision=lax.Precision.HIGHEST)


def _split_cols(hs_ref, hs):
    hs_ref[0] = hs[:, :DH]
    hs_ref[1] = hs[:, DH:]


def _tc_pre_body(phys_ref, pt_ref, degs_ref, degr_ref, pte_ref, w0a_ref,
                 w0b_ref, b0_ref, w1_ref, b1_ref, wm0_ref, bm0_ref, wm1_ref,
                 bm1_ref, x0_ref, hs_ref, invs_ref, invr_ref):
    pt = pt_ref[...][:, 0:1]
    iota = lax.broadcasted_iota(jnp.int32, (BM, 16), 1)
    onehot = (iota == pt).astype(jnp.float32)
    emb = _DOT(onehot, pte_ref[...])
    a = _DOT(phys_ref[...], w0a_ref[...]) + _DOT(emb, w0b_ref[...]) + b0_ref[...]
    x0 = _DOT(jnp.maximum(a, 0.0), w1_ref[...]) + b1_ref[...]
    invs = lax.rsqrt(jnp.maximum(degs_ref[...], 1.0))
    invr = lax.rsqrt(jnp.maximum(degr_ref[...], 1.0))
    h = jnp.maximum(_DOT(x0, wm0_ref[...]) + bm0_ref[...], 0.0)
    h = _DOT(h, wm1_ref[...]) + bm1_ref[...]
    x0_ref[...] = x0
    _split_cols(hs_ref, h * invs[:, 0:1])
    invs_ref[...] = invs
    invr_ref[...] = invr


_tc_pre = pl.pallas_call(
    _tc_pre_body,
    grid=(GRID,),
    in_specs=[
        pl.BlockSpec((BM, 32), lambda i: (i, 0)),
        pl.BlockSpec((BM, 8), lambda i: (i, 0)),
        pl.BlockSpec((BM, 8), lambda i: (i, 0)),
        pl.BlockSpec((BM, 8), lambda i: (i, 0)),
        pl.BlockSpec((16, 16), lambda i: (0, 0)),
        pl.BlockSpec((32, D), lambda i: (0, 0)),
        pl.BlockSpec((16, D), lambda i: (0, 0)),
        pl.BlockSpec((1, D), lambda i: (0, 0)),
        pl.BlockSpec((D, D), lambda i: (0, 0)),
        pl.BlockSpec((1, D), lambda i: (0, 0)),
        pl.BlockSpec((D, D), lambda i: (0, 0)),
        pl.BlockSpec((1, D), lambda i: (0, 0)),
        pl.BlockSpec((D, D), lambda i: (0, 0)),
        pl.BlockSpec((1, D), lambda i: (0, 0)),
    ],
    out_specs=[
        pl.BlockSpec((BM, D), lambda i: (i, 0)),
        pl.BlockSpec((NC, BM, DH), lambda i: (0, i, 0)),
        pl.BlockSpec((BM, 8), lambda i: (i, 0)),
        pl.BlockSpec((BM, 8), lambda i: (i, 0)),
    ],
    out_shape=[
        jax.ShapeDtypeStruct((NP, D), jnp.float32),
        jax.ShapeDtypeStruct((NC, NP, DH), jnp.float32),
        jax.ShapeDtypeStruct((NP, 8), jnp.float32),
        jax.ShapeDtypeStruct((NP, 8), jnp.float32),
    ],
)


def _tc_step_body(acc_ref, x_ref, invr_ref, invs_ref, w0_ref, b0_ref, w1_ref,
                  b1_ref, xn_ref, hs_ref):
    acc = jnp.concatenate([acc_ref[0], acc_ref[1]], axis=-1)
    xn = acc * invr_ref[...][:, 0:1] + x_ref[...]
    h = jnp.maximum(_DOT(xn, w0_ref[...]) + b0_ref[...], 0.0)
    h = _DOT(h, w1_ref[...]) + b1_ref[...]
    xn_ref[...] = xn
    _split_cols(hs_ref, h * invs_ref[...][:, 0:1])


_tc_step = pl.pallas_call(
    _tc_step_body,
    grid=(GRID,),
    in_specs=[
        pl.BlockSpec((NC, BM, DH), lambda i: (0, i, 0)),
        pl.BlockSpec((BM, D), lambda i: (i, 0)),
        pl.BlockSpec((BM, 8), lambda i: (i, 0)),
        pl.BlockSpec((BM, 8), lambda i: (i, 0)),
        pl.BlockSpec((D, D), lambda i: (0, 0)),
        pl.BlockSpec((1, D), lambda i: (0, 0)),
        pl.BlockSpec((D, D), lambda i: (0, 0)),
        pl.BlockSpec((1, D), lambda i: (0, 0)),
    ],
    out_specs=[
        pl.BlockSpec((BM, D), lambda i: (i, 0)),
        pl.BlockSpec((NC, BM, DH), lambda i: (0, i, 0)),
    ],
    out_shape=[
        jax.ShapeDtypeStruct((NP, D), jnp.float32),
        jax.ShapeDtypeStruct((NC, NP, DH), jnp.float32),
    ],
)


def _tc_post_body(acc_ref, x_ref, invr_ref, wp0_ref, bp0_ref, wp1_ref,
                  bp1_ref, out_ref):
    acc = jnp.concatenate([acc_ref[0], acc_ref[1]], axis=-1)
    xn = acc * invr_ref[...][:, 0:1] + x_ref[...]
    y = jnp.maximum(_DOT(xn, wp0_ref[...]) + bp0_ref[...], 0.0)
    out_ref[...] = _DOT(y, wp1_ref[...]) + bp1_ref[...]


_tc_post = pl.pallas_call(
    _tc_post_body,
    grid=(GRID,),
    in_specs=[
        pl.BlockSpec((NC, BM, DH), lambda i: (0, i, 0)),
        pl.BlockSpec((BM, D), lambda i: (i, 0)),
        pl.BlockSpec((BM, 8), lambda i: (i, 0)),
        pl.BlockSpec((D, D), lambda i: (0, 0)),
        pl.BlockSpec((1, D), lambda i: (0, 0)),
        pl.BlockSpec((D, D), lambda i: (0, 0)),
        pl.BlockSpec((1, D), lambda i: (0, 0)),
    ],
    out_specs=pl.BlockSpec((BM, D), lambda i: (i, 0)),
    out_shape=jax.ShapeDtypeStruct((NP, D), jnp.float32),
)


def kernel(vel_hist, vel_mag, bound, force, receivers, senders, particle_type,
           pt_embed, W_pre0, b_pre0, W_pre1, b_pre1, W_mp, b_mp,
           W_post0, b_post0, W_post1, b_post1):
    f32 = jnp.float32
    phys = jnp.concatenate([vel_hist, vel_mag, bound, force], axis=-1)
    phys = jnp.pad(phys, ((0, NP - N), (0, 3)))
    pt = jnp.pad(particle_type.astype(jnp.int32), (0, NP - N))
    pt8 = jnp.broadcast_to(pt[:, None], (NP, 8))
    pte = jnp.pad(pt_embed, ((0, 16 - pt_embed.shape[0]), (0, 0)))
    w0a = jnp.pad(W_pre0[:29], ((0, 3), (0, 0)))
    w0b = W_pre0[29:]
    wp1 = jnp.pad(W_post1, ((0, 0), (0, D - W_post1.shape[1])))
    bp1 = jnp.pad(b_post1, (0, D - b_post1.shape[0]))[None, :]

    def row(b):
        return b[None, :]

    dump = jnp.full((EPAD - E,), NP - 1, jnp.int32)
    send_p = jnp.concatenate([senders.astype(jnp.int32), dump]).reshape(NS, NB, B)
    recv_p = jnp.concatenate([receivers.astype(jnp.int32), dump]).reshape(NS, NB, B)

    ones8 = jnp.ones((B, 8), f32)
    zeros8 = jnp.zeros((ZR, 8), f32)
    zrows = jnp.zeros((ZR, DH), f32)

    deg_s, deg_r = _sc_deg(send_p, recv_p, ones8, zeros8)

    x, hs, invs, invr = _tc_pre(
        phys, pt8, deg_s, deg_r, pte, w0a, w0b, row(b_pre0), W_pre1,
        row(b_pre1), W_mp[0, 0], row(b_mp[0, 0]), W_mp[0, 1], row(b_mp[0, 1]))

    for step in range(NUM_MP):
        acc = _sc_agg(hs, send_p, recv_p, zrows)
        if step + 1 < NUM_MP:
            x, hs = _tc_step(acc, x, invr, invs, W_mp[step + 1, 0],
                             row(b_mp[step + 1, 0]), W_mp[step + 1, 1],
                             row(b_mp[step + 1, 1]))

    out = _tc_post(acc, x, invr, W_post0, row(b_post0), wp1, bp1)
    return out[:N, :3]


# speedup vs baseline: 1.5993x; 1.5993x over previous
import functools

import jax
import jax.numpy as jnp
from jax import lax
from jax.experimental import pallas as pl
from jax.experimental.pallas import tpu as pltpu
from jax.experimental.pallas import tpu_sc as plsc

N = 10000
NP = 10240
D = 128
DH = D // 2
E = 320000
NUM_MP = 10

NC = 2
NS = 16
B = 256
NB = 80
P = 20
NPH = NB // P
EPS = B * NB
EPAD = NS * EPS
ZR = NP // NS

BM = 2048
GRID = NP // BM

_MESH = plsc.VectorSubcoreMesh(
    core_axis_name="c", subcore_axis_name="s", num_cores=NC, num_subcores=NS)


def _sc_deg_body(send_hbm, recv_hbm, ones_hbm, zeros_hbm, out_s_hbm, out_r_hbm,
                 idx_v, ones_v, acc):
    c = lax.axis_index("c")
    s = lax.axis_index("s")

    @pl.when(c == 0)
    def _():
        pltpu.sync_copy(send_hbm.at[s], idx_v)

    @pl.when(c == 1)
    def _():
        pltpu.sync_copy(recv_hbm.at[s], idx_v)

    pltpu.sync_copy(ones_hbm, ones_v)
    pltpu.sync_copy(zeros_hbm, acc.at[pl.ds(s * ZR, ZR)])
    plsc.subcore_barrier()

    @pl.loop(0, NB)
    def _(j):
        pltpu.sync_copy(ones_v, acc.at[idx_v.at[j]], add=True)

    plsc.subcore_barrier()

    @pl.when(c == 0)
    def _():
        pltpu.sync_copy(acc.at[pl.ds(s * ZR, ZR)], out_s_hbm.at[pl.ds(s * ZR, ZR)])

    @pl.when(c == 1)
    def _():
        pltpu.sync_copy(acc.at[pl.ds(s * ZR, ZR)], out_r_hbm.at[pl.ds(s * ZR, ZR)])


_sc_deg = pl.kernel(
    _sc_deg_body,
    out_type=(jax.ShapeDtypeStruct((NP, 8), jnp.float32),
              jax.ShapeDtypeStruct((NP, 8), jnp.float32)),
    mesh=_MESH,
    scratch_types=[
        pltpu.VMEM((NB, B), jnp.int32),
        pltpu.VMEM((B, 8), jnp.float32),
        pltpu.VMEM_SHARED((NP, 8), jnp.float32),
    ],
    compiler_params=pltpu.CompilerParams(use_tc_tiling_on_sc=False),
)


def _sc_agg_body(h_hbm, send_hbm, recv_hbm, zeros_hbm, out_hbm,
                 sidx, ridx, rows_v, acc, sg0, sg1):
    gsems = (sg0, sg1)
    c = lax.axis_index("c")
    s = lax.axis_index("s")
    h_c = h_hbm.at[c]
    pltpu.sync_copy(zeros_hbm, acc.at[pl.ds(s * ZR, ZR)])
    plsc.subcore_barrier()

    for ph in range(NPH):
        pltpu.sync_copy(send_hbm.at[s, pl.ds(ph * P, P)], sidx)
        pltpu.sync_copy(recv_hbm.at[s, pl.ds(ph * P, P)], ridx)
        pltpu.async_copy(h_c.at[sidx.at[0]], rows_v.at[0], gsems[0])
        pltpu.async_copy(h_c.at[sidx.at[1]], rows_v.at[1], gsems[1])

        @pl.loop(0, P, step=2)
        def _(j0):
            for b in range(2):
                j = j0 + b
                pltpu.make_async_copy(h_c.at[sidx.at[j]], rows_v.at[b],
                                      gsems[b]).wait()
                pltpu.sync_copy(rows_v.at[b], acc.at[ridx.at[j]], add=True)

                @pl.when(j + 2 < P)
                def _():
                    pltpu.async_copy(h_c.at[sidx.at[j + 2]], rows_v.at[b],
                                     gsems[b])

    plsc.subcore_barrier()
    pltpu.sync_copy(acc.at[pl.ds(s * ZR, ZR)],
                    out_hbm.at[c, pl.ds(s * ZR, ZR)])


_sc_agg = pl.kernel(
    _sc_agg_body,
    out_type=jax.ShapeDtypeStruct((NC, NP, DH), jnp.bfloat16),
    mesh=_MESH,
    scratch_types=[
        pltpu.VMEM((P, B), jnp.int32),
        pltpu.VMEM((P, B), jnp.int32),
        pltpu.VMEM((2, B, DH), jnp.bfloat16),
        pltpu.VMEM_SHARED((NP, DH), jnp.bfloat16),
        pltpu.SemaphoreType.DMA,
        pltpu.SemaphoreType.DMA,
    ],
    compiler_params=pltpu.CompilerParams(use_tc_tiling_on_sc=False),
)


_DOT = functools.partial(jnp.dot, preferred_element_type=jnp.float32,
                         precision=lax.Precision.HIGHEST)


def _split_cols(hs_ref, hs):
    hs = hs.astype(jnp.bfloat16)
    hs_ref[0] = hs[:, :DH]
    hs_ref[1] = hs[:, DH:]


def _tc_pre_body(phys_ref, pt_ref, degs_ref, degr_ref, pte_ref, w0a_ref,
                 w0b_ref, b0_ref, w1_ref, b1_ref, wm0_ref, bm0_ref, wm1_ref,
                 bm1_ref, x0_ref, hs_ref, invs_ref, invr_ref):
    pt = pt_ref[...][:, 0:1]
    iota = lax.broadcasted_iota(jnp.int32, (BM, 16), 1)
    onehot = (iota == pt).astype(jnp.float32)
    emb = _DOT(onehot, pte_ref[...])
    a = _DOT(phys_ref[...], w0a_ref[...]) + _DOT(emb, w0b_ref[...]) + b0_ref[...]
    x0 = _DOT(jnp.maximum(a, 0.0), w1_ref[...]) + b1_ref[...]
    invs = lax.rsqrt(jnp.maximum(degs_ref[...], 1.0))
    invr = lax.rsqrt(jnp.maximum(degr_ref[...], 1.0))
    h = jnp.maximum(_DOT(x0, wm0_ref[...]) + bm0_ref[...], 0.0)
    h = _DOT(h, wm1_ref[...]) + bm1_ref[...]
    x0_ref[...] = x0
    _split_cols(hs_ref, h * invs[:, 0:1])
    invs_ref[...] = invs
    invr_ref[...] = invr


_tc_pre = pl.pallas_call(
    _tc_pre_body,
    grid=(GRID,),
    in_specs=[
        pl.BlockSpec((BM, 32), lambda i: (i, 0)),
        pl.BlockSpec((BM, 8), lambda i: (i, 0)),
        pl.BlockSpec((BM, 8), lambda i: (i, 0)),
        pl.BlockSpec((BM, 8), lambda i: (i, 0)),
        pl.BlockSpec((16, 16), lambda i: (0, 0)),
        pl.BlockSpec((32, D), lambda i: (0, 0)),
        pl.BlockSpec((16, D), lambda i: (0, 0)),
        pl.BlockSpec((1, D), lambda i: (0, 0)),
        pl.BlockSpec((D, D), lambda i: (0, 0)),
        pl.BlockSpec((1, D), lambda i: (0, 0)),
        pl.BlockSpec((D, D), lambda i: (0, 0)),
        pl.BlockSpec((1, D), lambda i: (0, 0)),
        pl.BlockSpec((D, D), lambda i: (0, 0)),
        pl.BlockSpec((1, D), lambda i: (0, 0)),
    ],
    out_specs=[
        pl.BlockSpec((BM, D), lambda i: (i, 0)),
        pl.BlockSpec((NC, BM, DH), lambda i: (0, i, 0)),
        pl.BlockSpec((BM, 8), lambda i: (i, 0)),
        pl.BlockSpec((BM, 8), lambda i: (i, 0)),
    ],
    out_shape=[
        jax.ShapeDtypeStruct((NP, D), jnp.float32),
        jax.ShapeDtypeStruct((NC, NP, DH), jnp.bfloat16),
        jax.ShapeDtypeStruct((NP, 8), jnp.float32),
        jax.ShapeDtypeStruct((NP, 8), jnp.float32),
    ],
)


def _tc_step_body(acc_ref, x_ref, invr_ref, invs_ref, w0_ref, b0_ref, w1_ref,
                  b1_ref, xn_ref, hs_ref):
    acc = jnp.concatenate([acc_ref[0], acc_ref[1]], axis=-1).astype(jnp.float32)
    xn = acc * invr_ref[...][:, 0:1] + x_ref[...]
    h = jnp.maximum(_DOT(xn, w0_ref[...]) + b0_ref[...], 0.0)
    h = _DOT(h, w1_ref[...]) + b1_ref[...]
    xn_ref[...] = xn
    _split_cols(hs_ref, h * invs_ref[...][:, 0:1])


_tc_step = pl.pallas_call(
    _tc_step_body,
    grid=(GRID,),
    in_specs=[
        pl.BlockSpec((NC, BM, DH), lambda i: (0, i, 0)),
        pl.BlockSpec((BM, D), lambda i: (i, 0)),
        pl.BlockSpec((BM, 8), lambda i: (i, 0)),
        pl.BlockSpec((BM, 8), lambda i: (i, 0)),
        pl.BlockSpec((D, D), lambda i: (0, 0)),
        pl.BlockSpec((1, D), lambda i: (0, 0)),
        pl.BlockSpec((D, D), lambda i: (0, 0)),
        pl.BlockSpec((1, D), lambda i: (0, 0)),
    ],
    out_specs=[
        pl.BlockSpec((BM, D), lambda i: (i, 0)),
        pl.BlockSpec((NC, BM, DH), lambda i: (0, i, 0)),
    ],
    out_shape=[
        jax.ShapeDtypeStruct((NP, D), jnp.float32),
        jax.ShapeDtypeStruct((NC, NP, DH), jnp.bfloat16),
    ],
)


def _tc_post_body(acc_ref, x_ref, invr_ref, wp0_ref, bp0_ref, wp1_ref,
                  bp1_ref, out_ref):
    acc = jnp.concatenate([acc_ref[0], acc_ref[1]], axis=-1).astype(jnp.float32)
    xn = acc * invr_ref[...][:, 0:1] + x_ref[...]
    y = jnp.maximum(_DOT(xn, wp0_ref[...]) + bp0_ref[...], 0.0)
    out_ref[...] = _DOT(y, wp1_ref[...]) + bp1_ref[...]


_tc_post = pl.pallas_call(
    _tc_post_body,
    grid=(GRID,),
    in_specs=[
        pl.BlockSpec((NC, BM, DH), lambda i: (0, i, 0)),
        pl.BlockSpec((BM, D), lambda i: (i, 0)),
        pl.BlockSpec((BM, 8), lambda i: (i, 0)),
        pl.BlockSpec((D, D), lambda i: (0, 0)),
        pl.BlockSpec((1, D), lambda i: (0, 0)),
        pl.BlockSpec((D, D), lambda i: (0, 0)),
        pl.BlockSpec((1, D), lambda i: (0, 0)),
    ],
    out_specs=pl.BlockSpec((BM, D), lambda i: (i, 0)),
    out_shape=jax.ShapeDtypeStruct((NP, D), jnp.float32),
)


def kernel(vel_hist, vel_mag, bound, force, receivers, senders, particle_type,
           pt_embed, W_pre0, b_pre0, W_pre1, b_pre1, W_mp, b_mp,
           W_post0, b_post0, W_post1, b_post1):
    f32 = jnp.float32
    phys = jnp.concatenate([vel_hist, vel_mag, bound, force], axis=-1)
    phys = jnp.pad(phys, ((0, NP - N), (0, 3)))
    pt = jnp.pad(particle_type.astype(jnp.int32), (0, NP - N))
    pt8 = jnp.broadcast_to(pt[:, None], (NP, 8))
    pte = jnp.pad(pt_embed, ((0, 16 - pt_embed.shape[0]), (0, 0)))
    w0a = jnp.pad(W_pre0[:29], ((0, 3), (0, 0)))
    w0b = W_pre0[29:]
    wp1 = jnp.pad(W_post1, ((0, 0), (0, D - W_post1.shape[1])))
    bp1 = jnp.pad(b_post1, (0, D - b_post1.shape[0]))[None, :]

    def row(b):
        return b[None, :]

    dump = jnp.full((EPAD - E,), NP - 1, jnp.int32)
    send_p = jnp.concatenate([senders.astype(jnp.int32), dump]).reshape(NS, NB, B)
    recv_p = jnp.concatenate([receivers.astype(jnp.int32), dump]).reshape(NS, NB, B)

    ones8 = jnp.ones((B, 8), f32)
    zeros8 = jnp.zeros((ZR, 8), f32)
    zrows = jnp.zeros((ZR, DH), jnp.bfloat16)

    deg_s, deg_r = _sc_deg(send_p, recv_p, ones8, zeros8)

    x, hs, invs, invr = _tc_pre(
        phys, pt8, deg_s, deg_r, pte, w0a, w0b, row(b_pre0), W_pre1,
        row(b_pre1), W_mp[0, 0], row(b_mp[0, 0]), W_mp[0, 1], row(b_mp[0, 1]))

    for step in range(NUM_MP):
        acc = _sc_agg(hs, send_p, recv_p, zrows)
        if step + 1 < NUM_MP:
            x, hs = _tc_step(acc, x, invr, invs, W_mp[step + 1, 0],
                             row(b_mp[step + 1, 0]), W_mp[step + 1, 1],
                             row(b_mp[step + 1, 1]))

    out = _tc_post(acc, x, invr, W_post0, row(b_post0), wp1, bp1)
    return out[:N, :3]


# speedup vs baseline: 1.6059x; 1.0042x over previous
import functools

import jax
import jax.numpy as jnp
from jax import lax
from jax.experimental import pallas as pl
from jax.experimental.pallas import tpu as pltpu
from jax.experimental.pallas import tpu_sc as plsc

N = 10000
NP = 10240
D = 128
DH = D // 2
E = 320000
NUM_MP = 10

NC = 2
NS = 16
B = 256
NB = 80
P = 20
NPH = NB // P
EPS = B * NB
EPAD = NS * EPS
ZR = NP // NS

BM = 2048
GRID = NP // BM

_MESH = plsc.VectorSubcoreMesh(
    core_axis_name="c", subcore_axis_name="s", num_cores=NC, num_subcores=NS)


def _sc_deg_body(send_hbm, recv_hbm, ones_hbm, zeros_hbm, out_s_hbm, out_r_hbm,
                 idx_v, ones_v, acc):
    c = lax.axis_index("c")
    s = lax.axis_index("s")

    @pl.when(c == 0)
    def _():
        pltpu.sync_copy(send_hbm.at[s], idx_v)

    @pl.when(c == 1)
    def _():
        pltpu.sync_copy(recv_hbm.at[s], idx_v)

    pltpu.sync_copy(ones_hbm, ones_v)
    pltpu.sync_copy(zeros_hbm, acc.at[pl.ds(s * ZR, ZR)])
    plsc.subcore_barrier()

    @pl.loop(0, NB)
    def _(j):
        pltpu.sync_copy(ones_v, acc.at[idx_v.at[j]], add=True)

    plsc.subcore_barrier()

    @pl.when(c == 0)
    def _():
        pltpu.sync_copy(acc.at[pl.ds(s * ZR, ZR)], out_s_hbm.at[pl.ds(s * ZR, ZR)])

    @pl.when(c == 1)
    def _():
        pltpu.sync_copy(acc.at[pl.ds(s * ZR, ZR)], out_r_hbm.at[pl.ds(s * ZR, ZR)])


_sc_deg = pl.kernel(
    _sc_deg_body,
    out_type=(jax.ShapeDtypeStruct((NP, 8), jnp.float32),
              jax.ShapeDtypeStruct((NP, 8), jnp.float32)),
    mesh=_MESH,
    scratch_types=[
        pltpu.VMEM((NB, B), jnp.int32),
        pltpu.VMEM((B, 8), jnp.float32),
        pltpu.VMEM_SHARED((NP, 8), jnp.float32),
    ],
    compiler_params=pltpu.CompilerParams(use_tc_tiling_on_sc=False),
)


def _sc_agg_body(h_hbm, send_hbm, recv_hbm, zeros_hbm, out_hbm,
                 sidx, ridx, rows_v, acc,
                 sg0, sg1, sg2, sg3, ss0, ss1, ss2, ss3):
    gsems = (sg0, sg1, sg2, sg3)
    ssems = (ss0, ss1, ss2, ss3)
    c = lax.axis_index("c")
    s = lax.axis_index("s")
    h_c = h_hbm.at[c]
    pltpu.sync_copy(zeros_hbm, acc.at[pl.ds(s * ZR, ZR)])
    plsc.subcore_barrier()

    for ph in range(NPH):
        pltpu.sync_copy(send_hbm.at[s, pl.ds(ph * P, P)], sidx)
        pltpu.sync_copy(recv_hbm.at[s, pl.ds(ph * P, P)], ridx)
        pltpu.async_copy(h_c.at[sidx.at[0]], rows_v.at[0], gsems[0])
        pltpu.async_copy(h_c.at[sidx.at[1]], rows_v.at[1], gsems[1])

        @pl.loop(0, P, step=4)
        def _(j0):
            for bi in range(4):
                j = j0 + bi
                pltpu.make_async_copy(h_c.at[sidx.at[j]], rows_v.at[bi],
                                      gsems[bi]).wait()
                pltpu.async_copy(rows_v.at[bi], acc.at[ridx.at[j]], ssems[bi],
                                 add=True)
                nb = (bi + 2) % 4

                @pl.when(j + 2 < P)
                def _():
                    @pl.when(j >= 2)
                    def _():
                        pltpu.make_async_copy(rows_v.at[nb],
                                              acc.at[ridx.at[j]],
                                              ssems[nb]).wait()

                    pltpu.async_copy(h_c.at[sidx.at[j + 2]], rows_v.at[nb],
                                     gsems[nb])

        for bi in range(4):
            pltpu.make_async_copy(rows_v.at[bi], acc.at[ridx.at[0]],
                                  ssems[bi]).wait()

    plsc.subcore_barrier()
    pltpu.sync_copy(acc.at[pl.ds(s * ZR, ZR)],
                    out_hbm.at[c, pl.ds(s * ZR, ZR)])


_sc_agg = pl.kernel(
    _sc_agg_body,
    out_type=jax.ShapeDtypeStruct((NC, NP, DH), jnp.bfloat16),
    mesh=_MESH,
    scratch_types=[
        pltpu.VMEM((P, B), jnp.int32),
        pltpu.VMEM((P, B), jnp.int32),
        pltpu.VMEM((4, B, DH), jnp.bfloat16),
        pltpu.VMEM_SHARED((NP, DH), jnp.bfloat16),
        pltpu.SemaphoreType.DMA,
        pltpu.SemaphoreType.DMA,
        pltpu.SemaphoreType.DMA,
        pltpu.SemaphoreType.DMA,
        pltpu.SemaphoreType.DMA,
        pltpu.SemaphoreType.DMA,
        pltpu.SemaphoreType.DMA,
        pltpu.SemaphoreType.DMA,
    ],
    compiler_params=pltpu.CompilerParams(use_tc_tiling_on_sc=False),
)


_DOT = functools.partial(jnp.dot, preferred_element_type=jnp.float32,
                         precision=lax.Precision.HIGHEST)


def _split_cols(hs_ref, hs):
    hs = hs.astype(jnp.bfloat16)
    hs_ref[0] = hs[:, :DH]
    hs_ref[1] = hs[:, DH:]


def _tc_pre_body(phys_ref, pt_ref, degs_ref, degr_ref, pte_ref, w0a_ref,
                 w0b_ref, b0_ref, w1_ref, b1_ref, wm0_ref, bm0_ref, wm1_ref,
                 bm1_ref, x0_ref, hs_ref, invs_ref, invr_ref):
    pt = pt_ref[...][:, 0:1]
    iota = lax.broadcasted_iota(jnp.int32, (BM, 16), 1)
    onehot = (iota == pt).astype(jnp.float32)
    emb = _DOT(onehot, pte_ref[...])
    a = _DOT(phys_ref[...], w0a_ref[...]) + _DOT(emb, w0b_ref[...]) + b0_ref[...]
    x0 = _DOT(jnp.maximum(a, 0.0), w1_ref[...]) + b1_ref[...]
    invs = lax.rsqrt(jnp.maximum(degs_ref[...], 1.0))
    invr = lax.rsqrt(jnp.maximum(degr_ref[...], 1.0))
    h = jnp.maximum(_DOT(x0, wm0_ref[...]) + bm0_ref[...], 0.0)
    h = _DOT(h, wm1_ref[...]) + bm1_ref[...]
    x0_ref[...] = x0
    _split_cols(hs_ref, h * invs[:, 0:1])
    invs_ref[...] = invs
    invr_ref[...] = invr


_tc_pre = pl.pallas_call(
    _tc_pre_body,
    grid=(GRID,),
    in_specs=[
        pl.BlockSpec((BM, 32), lambda i: (i, 0)),
        pl.BlockSpec((BM, 8), lambda i: (i, 0)),
        pl.BlockSpec((BM, 8), lambda i: (i, 0)),
        pl.BlockSpec((BM, 8), lambda i: (i, 0)),
        pl.BlockSpec((16, 16), lambda i: (0, 0)),
        pl.BlockSpec((32, D), lambda i: (0, 0)),
        pl.BlockSpec((16, D), lambda i: (0, 0)),
        pl.BlockSpec((1, D), lambda i: (0, 0)),
        pl.BlockSpec((D, D), lambda i: (0, 0)),
        pl.BlockSpec((1, D), lambda i: (0, 0)),
        pl.BlockSpec((D, D), lambda i: (0, 0)),
        pl.BlockSpec((1, D), lambda i: (0, 0)),
        pl.BlockSpec((D, D), lambda i: (0, 0)),
        pl.BlockSpec((1, D), lambda i: (0, 0)),
    ],
    out_specs=[
        pl.BlockSpec((BM, D), lambda i: (i, 0)),
        pl.BlockSpec((NC, BM, DH), lambda i: (0, i, 0)),
        pl.BlockSpec((BM, 8), lambda i: (i, 0)),
        pl.BlockSpec((BM, 8), lambda i: (i, 0)),
    ],
    out_shape=[
        jax.ShapeDtypeStruct((NP, D), jnp.float32),
        jax.ShapeDtypeStruct((NC, NP, DH), jnp.bfloat16),
        jax.ShapeDtypeStruct((NP, 8), jnp.float32),
        jax.ShapeDtypeStruct((NP, 8), jnp.float32),
    ],
)


def _tc_step_body(acc_ref, x_ref, invr_ref, invs_ref, w0_ref, b0_ref, w1_ref,
                  b1_ref, xn_ref, hs_ref):
    acc = jnp.concatenate([acc_ref[0], acc_ref[1]], axis=-1).astype(jnp.float32)
    xn = acc * invr_ref[...][:, 0:1] + x_ref[...]
    h = jnp.maximum(_DOT(xn, w0_ref[...]) + b0_ref[...], 0.0)
    h = _DOT(h, w1_ref[...]) + b1_ref[...]
    xn_ref[...] = xn
    _split_cols(hs_ref, h * invs_ref[...][:, 0:1])


_tc_step = pl.pallas_call(
    _tc_step_body,
    grid=(GRID,),
    in_specs=[
        pl.BlockSpec((NC, BM, DH), lambda i: (0, i, 0)),
        pl.BlockSpec((BM, D), lambda i: (i, 0)),
        pl.BlockSpec((BM, 8), lambda i: (i, 0)),
        pl.BlockSpec((BM, 8), lambda i: (i, 0)),
        pl.BlockSpec((D, D), lambda i: (0, 0)),
        pl.BlockSpec((1, D), lambda i: (0, 0)),
        pl.BlockSpec((D, D), lambda i: (0, 0)),
        pl.BlockSpec((1, D), lambda i: (0, 0)),
    ],
    out_specs=[
        pl.BlockSpec((BM, D), lambda i: (i, 0)),
        pl.BlockSpec((NC, BM, DH), lambda i: (0, i, 0)),
    ],
    out_shape=[
        jax.ShapeDtypeStruct((NP, D), jnp.float32),
        jax.ShapeDtypeStruct((NC, NP, DH), jnp.bfloat16),
    ],
)


def _tc_post_body(acc_ref, x_ref, invr_ref, wp0_ref, bp0_ref, wp1_ref,
                  bp1_ref, out_ref):
    acc = jnp.concatenate([acc_ref[0], acc_ref[1]], axis=-1).astype(jnp.float32)
    xn = acc * invr_ref[...][:, 0:1] + x_ref[...]
    y = jnp.maximum(_DOT(xn, wp0_ref[...]) + bp0_ref[...], 0.0)
    out_ref[...] = _DOT(y, wp1_ref[...]) + bp1_ref[...]


_tc_post = pl.pallas_call(
    _tc_post_body,
    grid=(GRID,),
    in_specs=[
        pl.BlockSpec((NC, BM, DH), lambda i: (0, i, 0)),
        pl.BlockSpec((BM, D), lambda i: (i, 0)),
        pl.BlockSpec((BM, 8), lambda i: (i, 0)),
        pl.BlockSpec((D, D), lambda i: (0, 0)),
        pl.BlockSpec((1, D), lambda i: (0, 0)),
        pl.BlockSpec((D, D), lambda i: (0, 0)),
        pl.BlockSpec((1, D), lambda i: (0, 0)),
    ],
    out_specs=pl.BlockSpec((BM, D), lambda i: (i, 0)),
    out_shape=jax.ShapeDtypeStruct((NP, D), jnp.float32),
)


def kernel(vel_hist, vel_mag, bound, force, receivers, senders, particle_type,
           pt_embed, W_pre0, b_pre0, W_pre1, b_pre1, W_mp, b_mp,
           W_post0, b_post0, W_post1, b_post1):
    f32 = jnp.float32
    phys = jnp.concatenate([vel_hist, vel_mag, bound, force], axis=-1)
    phys = jnp.pad(phys, ((0, NP - N), (0, 3)))
    pt = jnp.pad(particle_type.astype(jnp.int32), (0, NP - N))
    pt8 = jnp.broadcast_to(pt[:, None], (NP, 8))
    pte = jnp.pad(pt_embed, ((0, 16 - pt_embed.shape[0]), (0, 0)))
    w0a = jnp.pad(W_pre0[:29], ((0, 3), (0, 0)))
    w0b = W_pre0[29:]
    wp1 = jnp.pad(W_post1, ((0, 0), (0, D - W_post1.shape[1])))
    bp1 = jnp.pad(b_post1, (0, D - b_post1.shape[0]))[None, :]

    def row(b):
        return b[None, :]

    dump = jnp.full((EPAD - E,), NP - 1, jnp.int32)
    send_p = jnp.concatenate([senders.astype(jnp.int32), dump]).reshape(NS, NB, B)
    recv_p = jnp.concatenate([receivers.astype(jnp.int32), dump]).reshape(NS, NB, B)

    ones8 = jnp.ones((B, 8), f32)
    zeros8 = jnp.zeros((ZR, 8), f32)
    zrows = jnp.zeros((ZR, DH), jnp.bfloat16)

    deg_s, deg_r = _sc_deg(send_p, recv_p, ones8, zeros8)

    x, hs, invs, invr = _tc_pre(
        phys, pt8, deg_s, deg_r, pte, w0a, w0b, row(b_pre0), W_pre1,
        row(b_pre1), W_mp[0, 0], row(b_mp[0, 0]), W_mp[0, 1], row(b_mp[0, 1]))

    for step in range(NUM_MP):
        acc = _sc_agg(hs, send_p, recv_p, zrows)
        if step + 1 < NUM_MP:
            x, hs = _tc_step(acc, x, invr, invs, W_mp[step + 1, 0],
                             row(b_mp[step + 1, 0]), W_mp[step + 1, 1],
                             row(b_mp[step + 1, 1]))

    out = _tc_post(acc, x, invr, W_post0, row(b_post0), wp1, bp1)
    return out[:N, :3]
